# baseline (speedup 1.0000x reference)
import functools

import jax
import jax.numpy as jnp
from jax import lax
from jax.experimental import pallas as pl
from jax.experimental.pallas import tpu as pltpu
from jax.experimental.pallas import tpu_sc as plsc

B = 8
C = 128
V = 10000
E = 320000
K = 3

NC = 2
NS = 16
BPC = B // NC
LANES = 16

CH = 112
NCH = 180
EPT = NCH * CH
EPAD = NS * EPT
NR = 3
NE = 6

RPT = 624
ZR = 8
REM = V - NS * RPT


def _spmm_sc(x_flat, e3):
    mesh = plsc.VectorSubcoreMesh(core_axis_name="c", subcore_axis_name="s")

    @functools.partial(
        pl.kernel,
        mesh=mesh,
        out_type=jax.ShapeDtypeStruct((B * V, C), jnp.float32),
        scratch_types=(
            [pltpu.VMEM((3, CH), jnp.int32) for _ in range(NE)]
            + [pltpu.VMEM((CH, C), jnp.float32) for _ in range(NR)]
            + [
                pltpu.VMEM((ZR, C), jnp.float32),
                pltpu.VMEM_SHARED((V, C), jnp.float32),
                pltpu.SemaphoreType.DMA((NE,)),
                pltpu.SemaphoreType.DMA((NR,)),
                pltpu.SemaphoreType.DMA((NR,)),
            ]
        ),
    )
    def k(x_hbm, e3_hbm, y_hbm, *scratch):
        esl = scratch[0:NE]
        rows = scratch[NE:NE + NR]
        zbuf, acc, esem, gsem, ssem = scratch[NE + NR:]

        c = lax.axis_index("c")
        s = lax.axis_index("s")

        zeros16 = jnp.zeros((LANES,), jnp.float32)

        def zb_body(i, carry):
            for t in range(C // LANES):
                zbuf[i, pl.ds(t * LANES, LANES)] = zeros16
            return carry

        lax.fori_loop(0, ZR, zb_body, 0)

        def fire_edges(i, q):
            pltpu.async_copy(e3_hbm.at[s].at[i], esl[q], esem.at[q])

        def wait_edges(q):
            pltpu.make_async_copy(e3_hbm.at[s].at[0], esl[q],
                                  esem.at[q]).wait()

        def fire_gather(q, p, boff):
            for t in range(CH // LANES):
                sl = pl.ds(t * LANES, LANES)
                esl[q][0, sl] = esl[q][0, sl] + boff
            pltpu.async_copy(x_hbm.at[esl[q].at[0]], rows[p], gsem.at[p])

        def wait_gather(q, p):
            pltpu.make_async_copy(x_hbm.at[esl[q].at[0]], rows[p],
                                  gsem.at[p]).wait()

        def wait_scatter(p):
            pltpu.make_async_copy(rows[p], acc.at[pl.ds(0, CH)],
                                  ssem.at[p]).wait()

        def batch_body(jb, carry):
            boff = (c * BPC + jb) * V

            fire_edges(0, 0)
            fire_edges(1, 1)

            for q in range(RPT // ZR):
                pltpu.sync_copy(zbuf, acc.at[pl.ds(s * RPT + q * ZR, ZR)])

            @pl.when(s == NS - 1)
            def _():
                for q in range(REM // ZR):
                    pltpu.sync_copy(zbuf,
                                    acc.at[pl.ds(NS * RPT + q * ZR, ZR)])

            wait_edges(0)
            fire_gather(0, 0, boff)
            plsc.subcore_barrier()

            def do_chunk(i, i6, u):
                p = u % NR
                q = u % NE
                pn = (u + 1) % NR
                qn = (u + 1) % NE
                qf = (u + 2) % NE

                if u <= 1:
                    @pl.when(i6 > 0)
                    def _():
                        wait_scatter((u + 1) % NR)
                else:
                    wait_scatter((u + 1) % NR)

                def prefetch_edges():
                    fire_edges(i + 2, qf)

                if u < NE - 2:
                    prefetch_edges()
                else:
                    pl.when(i6 < NCH // NE - 1)(prefetch_edges)

                def next_gather():
                    wait_edges(qn)
                    fire_gather(qn, pn, boff)

                if u < NE - 1:
                    next_gather()
                else:
                    pl.when(i6 < NCH // NE - 1)(next_gather)

                wait_gather(q, p)

                def scale(g, carry3):
                    wg = lax.bitcast_convert_type(
                        esl[q][2, pl.ds(g * LANES, LANES)], jnp.float32)
                    base_r = g * LANES
                    for r16 in range(LANES):
                        wr = wg[r16]
                        for t in range(C // LANES):
                            sl = pl.ds(t * LANES, LANES)
                            rows[p][base_r + r16, sl] = \
                                rows[p][base_r + r16, sl] * wr
                    return carry3

                lax.fori_loop(0, CH // LANES, scale, 0)
                pltpu.async_copy(rows[p], acc.at[esl[q].at[1]],
                                 ssem.at[p], add=True)

            def chunk_six(i6, carry2):
                for u in range(NE):
                    do_chunk(NE * i6 + u, i6, u)
                return carry2

            lax.fori_loop(0, NCH // NE, chunk_six, 0)
            wait_scatter((NCH - 2) % NR)
            wait_scatter((NCH - 1) % NR)
            plsc.subcore_barrier()

            pltpu.sync_copy(acc.at[pl.ds(s * RPT, RPT)],
                            y_hbm.at[pl.ds(boff + s * RPT, RPT)])

            @pl.when(s == NS - 1)
            def _():
                pltpu.sync_copy(acc.at[pl.ds(NS * RPT, REM)],
                                y_hbm.at[pl.ds(boff + NS * RPT, REM)])

            plsc.subcore_barrier()
            return carry

        lax.fori_loop(0, BPC, batch_body, 0)

    return k(x_flat, e3)


def _combine_tc(x0, x1, z2, w3, bias2d):
    VT = 512
    nj = pl.cdiv(V, VT)

    def body(x0_ref, x1_ref, z2_ref, w_ref, b_ref, o_ref):
        dn = (((0,), (1,)), ((), ()))
        acc = lax.dot_general(w_ref[0], x0_ref[0], dn,
                              preferred_element_type=jnp.float32)
        acc += lax.dot_general(w_ref[1], x1_ref[0], dn,
                               preferred_element_type=jnp.float32)
        acc += lax.dot_general(w_ref[2], z2_ref[0], dn,
                               preferred_element_type=jnp.float32)
        o_ref[0] = acc + b_ref[...]

    xspec = pl.BlockSpec((1, VT, C), lambda b, j: (b, j, 0))
    return pl.pallas_call(
        body,
        grid=(B, nj),
        in_specs=[
            xspec, xspec, xspec,
            pl.BlockSpec((K, C, C), lambda b, j: (0, 0, 0)),
            pl.BlockSpec((C, 1), lambda b, j: (0, 0)),
        ],
        out_specs=pl.BlockSpec((1, C, VT), lambda b, j: (b, 0, j)),
        out_shape=jax.ShapeDtypeStruct((B, C, V), jnp.float32),
    )(x0, x1, z2, w3, bias2d)


def kernel(x, edge_index, edge_weight, weight, bias):
    xp = jnp.transpose(x, (0, 2, 1)).reshape(B * V, C)

    pad = EPAD - E
    src3 = jnp.concatenate(
        [edge_index[0], jnp.zeros((pad,), jnp.int32)]).reshape(NS, NCH, CH)
    dst3 = jnp.concatenate(
        [edge_index[1], jnp.zeros((pad,), jnp.int32)]).reshape(NS, NCH, CH)
    wbits = lax.bitcast_convert_type(
        jnp.concatenate([edge_weight, jnp.zeros((pad,), jnp.float32)]),
        jnp.int32).reshape(NS, NCH, CH)
    e3 = jnp.stack([src3, dst3, wbits], axis=2)

    x1 = _spmm_sc(xp, e3)
    z2 = _spmm_sc(x1, e3)

    wk = weight.reshape(C, K, C)
    w3 = jnp.stack([wk[:, 0, :] - wk[:, 2, :],
                    wk[:, 1, :],
                    2.0 * wk[:, 2, :]], axis=0)

    return _combine_tc(xp.reshape(B, V, C),
                       x1.reshape(B, V, C),
                       z2.reshape(B, V, C),
                       w3, bias[:, None])


# speedup vs baseline: 1.0169x; 1.0169x over previous
import functools

import jax
import jax.numpy as jnp
from jax import lax
from jax.experimental import pallas as pl
from jax.experimental.pallas import tpu as pltpu
from jax.experimental.pallas import tpu_sc as plsc

B = 8
C = 128
V = 10000
E = 320000
K = 3

NC = 2
NS = 16
BPC = B // NC
LANES = 16

CH = 112
NCH = 180
EPT = NCH * CH
EPAD = NS * EPT
NR = 3
NE = 6

RPT = 624
ZR = 16
REM = V - NS * RPT


def _spmm_sc(x_flat, src3, dst3, w3e):
    mesh = plsc.VectorSubcoreMesh(core_axis_name="c", subcore_axis_name="s")

    @functools.partial(
        pl.kernel,
        mesh=mesh,
        out_type=jax.ShapeDtypeStruct((B * V, C), jnp.float32),
        scratch_types=(
            [pltpu.VMEM((CH,), jnp.int32) for _ in range(NE)]
            + [pltpu.VMEM((CH,), jnp.int32) for _ in range(NE)]
            + [pltpu.VMEM((CH,), jnp.float32) for _ in range(NE)]
            + [pltpu.VMEM((CH, C), jnp.float32) for _ in range(NR)]
            + [
                pltpu.VMEM((ZR, C), jnp.float32),
                pltpu.VMEM_SHARED((V, C), jnp.float32),
                pltpu.SemaphoreType.DMA((NE,)),
                pltpu.SemaphoreType.DMA((NR,)),
                pltpu.SemaphoreType.DMA((NR,)),
            ]
        ),
    )
    def k(x_hbm, src_hbm, dst_hbm, w_hbm, y_hbm, *scratch):
        sidx = scratch[0:NE]
        didx = scratch[NE:2 * NE]
        wv = scratch[2 * NE:3 * NE]
        rows = scratch[3 * NE:3 * NE + NR]
        zbuf, acc, esem, gsem, ssem = scratch[3 * NE + NR:]

        c = lax.axis_index("c")
        s = lax.axis_index("s")

        zeros16 = jnp.zeros((LANES,), jnp.float32)

        def zb_body(i, carry):
            for t in range(C // LANES):
                zbuf[i, pl.ds(t * LANES, LANES)] = zeros16
            return carry

        lax.fori_loop(0, ZR, zb_body, 0)

        def fire_edges(i, q):
            pltpu.async_copy(src_hbm.at[s].at[i], sidx[q], esem.at[q])
            pltpu.async_copy(dst_hbm.at[s].at[i], didx[q], esem.at[q])
            pltpu.async_copy(w_hbm.at[s].at[i], wv[q], esem.at[q])

        def wait_edges(q):
            pltpu.make_async_copy(src_hbm.at[s].at[0], sidx[q],
                                  esem.at[q]).wait()
            pltpu.make_async_copy(dst_hbm.at[s].at[0], didx[q],
                                  esem.at[q]).wait()
            pltpu.make_async_copy(w_hbm.at[s].at[0], wv[q],
                                  esem.at[q]).wait()

        def fire_gather(q, p, boff):
            for t in range(CH // LANES):
                sl = pl.ds(t * LANES, LANES)
                sidx[q][sl] = sidx[q][sl] + boff
            pltpu.async_copy(x_hbm.at[sidx[q]], rows[p], gsem.at[p])

        def wait_gather(q, p):
            pltpu.make_async_copy(x_hbm.at[sidx[q]], rows[p],
                                  gsem.at[p]).wait()

        def wait_scatter(p):
            pltpu.make_async_copy(rows[p], acc.at[pl.ds(0, CH)],
                                  ssem.at[p]).wait()

        def batch_body(jb, carry):
            boff = (c * BPC + jb) * V

            fire_edges(0, 0)
            fire_edges(1, 1)

            for q in range(RPT // ZR):
                pltpu.sync_copy(zbuf, acc.at[pl.ds(s * RPT + q * ZR, ZR)])

            @pl.when(s == NS - 1)
            def _():
                pltpu.sync_copy(zbuf, acc.at[pl.ds(NS * RPT, REM)])

            wait_edges(0)
            fire_gather(0, 0, boff)
            plsc.subcore_barrier()

            def do_chunk(i, i6, u):
                p = u % NR
                q = u % NE
                pn = (u + 1) % NR
                qn = (u + 1) % NE
                qf = (u + 2) % NE

                if u <= 1:
                    @pl.when(i6 > 0)
                    def _():
                        wait_scatter((u + 1) % NR)
                else:
                    wait_scatter((u + 1) % NR)

                def prefetch_edges():
                    fire_edges(i + 2, qf)

                if u < NE - 2:
                    prefetch_edges()
                else:
                    pl.when(i6 < NCH // NE - 1)(prefetch_edges)

                def next_gather():
                    wait_edges(qn)
                    fire_gather(qn, pn, boff)

                if u < NE - 1:
                    next_gather()
                else:
                    pl.when(i6 < NCH // NE - 1)(next_gather)

                wait_gather(q, p)

                def scale(g, carry3):
                    wg = wv[q][pl.ds(g * LANES, LANES)]
                    base_r = g * LANES
                    for r16 in range(LANES):
                        wr = wg[r16]
                        for t in range(C // LANES):
                            sl = pl.ds(t * LANES, LANES)
                            rows[p][base_r + r16, sl] = \
                                rows[p][base_r + r16, sl] * wr
                    return carry3

                lax.fori_loop(0, CH // LANES, scale, 0)
                pltpu.async_copy(rows[p], acc.at[didx[q]],
                                 ssem.at[p], add=True)

            def chunk_six(i6, carry2):
                for u in range(NE):
                    do_chunk(NE * i6 + u, i6, u)
                return carry2

            lax.fori_loop(0, NCH // NE, chunk_six, 0)
            wait_scatter((NCH - 2) % NR)
            wait_scatter((NCH - 1) % NR)
            plsc.subcore_barrier()

            pltpu.sync_copy(acc.at[pl.ds(s * RPT, RPT)],
                            y_hbm.at[pl.ds(boff + s * RPT, RPT)])

            @pl.when(s == NS - 1)
            def _():
                pltpu.sync_copy(acc.at[pl.ds(NS * RPT, REM)],
                                y_hbm.at[pl.ds(boff + NS * RPT, REM)])

            plsc.subcore_barrier()
            return carry

        lax.fori_loop(0, BPC, batch_body, 0)

    return k(x_flat, src3, dst3, w3e)


def _combine_tc(x0, x1, z2, w3, bias2d):
    VT = 512
    nj = pl.cdiv(V, VT)

    def body(x0_ref, x1_ref, z2_ref, w_ref, b_ref, o_ref):
        dn = (((0,), (1,)), ((), ()))
        acc = lax.dot_general(w_ref[0], x0_ref[0], dn,
                              preferred_element_type=jnp.float32)
        acc += lax.dot_general(w_ref[1], x1_ref[0], dn,
                               preferred_element_type=jnp.float32)
        acc += lax.dot_general(w_ref[2], z2_ref[0], dn,
                               preferred_element_type=jnp.float32)
        o_ref[0] = acc + b_ref[...]

    xspec = pl.BlockSpec((1, VT, C), lambda b, j: (b, j, 0))
    return pl.pallas_call(
        body,
        grid=(B, nj),
        in_specs=[
            xspec, xspec, xspec,
            pl.BlockSpec((K, C, C), lambda b, j: (0, 0, 0)),
            pl.BlockSpec((C, 1), lambda b, j: (0, 0)),
        ],
        out_specs=pl.BlockSpec((1, C, VT), lambda b, j: (b, 0, j)),
        out_shape=jax.ShapeDtypeStruct((B, C, V), jnp.float32),
    )(x0, x1, z2, w3, bias2d)


def kernel(x, edge_index, edge_weight, weight, bias):
    xp = jnp.transpose(x, (0, 2, 1)).reshape(B * V, C)

    pad = EPAD - E
    src3 = jnp.concatenate(
        [edge_index[0], jnp.zeros((pad,), jnp.int32)]).reshape(NS, NCH, CH)
    dst3 = jnp.concatenate(
        [edge_index[1], jnp.zeros((pad,), jnp.int32)]).reshape(NS, NCH, CH)
    w3e = jnp.concatenate(
        [edge_weight, jnp.zeros((pad,), jnp.float32)]).reshape(NS, NCH, CH)

    x1 = _spmm_sc(xp, src3, dst3, w3e)
    z2 = _spmm_sc(x1, src3, dst3, w3e)

    wk = weight.reshape(C, K, C)
    w3 = jnp.stack([wk[:, 0, :] - wk[:, 2, :],
                    wk[:, 1, :],
                    2.0 * wk[:, 2, :]], axis=0)

    return _combine_tc(xp.reshape(B, V, C),
                       x1.reshape(B, V, C),
                       z2.reshape(B, V, C),
                       w3, bias[:, None])
